# Initial kernel scaffold; baseline (speedup 1.0000x reference)
import jax
import jax.numpy as jnp
from jax.experimental import pallas as pl


def kernel(feature, A, W1, b1, W2, b2, temp):
    raise NotImplementedError("write your pallas kernel here")


# speedup vs baseline: 3.3327x; 3.3327x over previous
import functools

import jax
import jax.numpy as jnp
from jax import lax
from jax.experimental import pallas as pl
from jax.experimental.pallas import tpu as pltpu


def _pick_block(n, candidates):
    for c in candidates:
        if n % c == 0:
            return c
    return n


def _cast_deg_body(a_ref, abf_ref, dinv_ref):
    a = a_ref[...]
    abf_ref[...] = a.astype(jnp.bfloat16)
    deg = jnp.sum(a, axis=1, keepdims=True)
    dinv_ref[...] = jnp.where(deg > 0.0, lax.rsqrt(deg), 0.0)


def _mlp_body(f_ref, w1_ref, b1_ref, w2_ref, b2_ref, x_ref):
    h = jnp.dot(f_ref[...], w1_ref[...], preferred_element_type=jnp.float32)
    h = jnp.maximum(h + b1_ref[...], 0.0)
    x_ref[...] = (
        jnp.dot(h, w2_ref[...], preferred_element_type=jnp.float32) + b2_ref[...]
    )


def _cheb_body(nb, coe_ref, abf_ref, x_ref, dinv_ref, out_ref,
               acc_ref, tx0_ref, tx1_ref, ybf_ref):
    k = pl.program_id(0)
    rb = pl.program_id(1)

    @pl.when(rb == 0)
    def _prologue():
        @pl.when(k == 0)
        def _init():
            tx0_ref[...] = x_ref[...]
            out_ref[...] = (coe_ref[0] * 0.5) * x_ref[...]

        h = jnp.where(k == 0, x_ref[...], tx1_ref[...])
        ybf_ref[...] = (dinv_ref[...] * h).astype(jnp.bfloat16)

    rbsz = abf_ref.shape[0]
    yblk = ybf_ref[pl.ds(rb * rbsz, rbsz), :]
    contrib = lax.dot_general(
        abf_ref[...], yblk, (((0,), (0,)), ((), ())),
        preferred_element_type=jnp.float32)

    @pl.when(rb == 0)
    def _acc_init():
        acc_ref[...] = contrib

    @pl.when(rb > 0)
    def _acc_add():
        acc_ref[...] = acc_ref[...] + contrib

    @pl.when(rb == nb - 1)
    def _epilogue():
        prop = -dinv_ref[...] * acc_ref[...]
        txn = jnp.where(k == 0, prop, 2.0 * prop - tx0_ref[...])
        out_ref[...] = out_ref[...] + coe_ref[k + 1] * txn
        tx0_ref[...] = jnp.where(k == 0, x_ref[...], tx1_ref[...])
        tx1_ref[...] = txn


@jax.jit
def kernel(feature, A, W1, b1, W2, b2, temp):
    n = A.shape[0]
    nfeat = feature.shape[1]
    nhid = W1.shape[1]
    f = W2.shape[1]
    kk = temp.shape[0] - 1

    ct = jax.nn.relu(temp)
    j = jnp.arange(kk + 1, dtype=jnp.float32)
    xj = jnp.cos((kk - j + 0.5) * jnp.pi / (kk + 1))
    i = jnp.arange(kk + 1, dtype=jnp.float32)
    tcheb = jnp.cos(i[:, None] * jnp.arccos(xj)[None, :])
    coe = (2.0 / (kk + 1)) * (tcheb @ ct)

    rb1 = _pick_block(n, (200, 100, 50, 40, 25, 10, 8))
    abf, dinv = pl.pallas_call(
        _cast_deg_body,
        grid=(n // rb1,),
        in_specs=[pl.BlockSpec((rb1, n), lambda i: (i, 0))],
        out_specs=[pl.BlockSpec((rb1, n), lambda i: (i, 0)),
                   pl.BlockSpec((rb1, 1), lambda i: (i, 0))],
        out_shape=[jax.ShapeDtypeStruct((n, n), jnp.bfloat16),
                   jax.ShapeDtypeStruct((n, 1), jnp.float32)],
    )(A)

    mb = _pick_block(n, (1000, 500, 200, 100, 50, 10, 8))
    x = pl.pallas_call(
        _mlp_body,
        grid=(n // mb,),
        in_specs=[pl.BlockSpec((mb, nfeat), lambda i: (i, 0)),
                  pl.BlockSpec((nfeat, nhid), lambda i: (0, 0)),
                  pl.BlockSpec((1, nhid), lambda i: (0, 0)),
                  pl.BlockSpec((nhid, f), lambda i: (0, 0)),
                  pl.BlockSpec((1, f), lambda i: (0, 0))],
        out_specs=pl.BlockSpec((mb, f), lambda i: (i, 0)),
        out_shape=jax.ShapeDtypeStruct((n, f), jnp.float32),
    )(feature, W1, b1.reshape(1, nhid), W2, b2.reshape(1, f))

    rb = _pick_block(n, (400, 200, 100, 50, 40, 25, 10, 8))
    nb = n // rb
    out = pl.pallas_call(
        functools.partial(_cheb_body, nb),
        grid=(kk, nb),
        in_specs=[pl.BlockSpec(memory_space=pltpu.SMEM),
                  pl.BlockSpec((rb, n), lambda k, r: (r, 0)),
                  pl.BlockSpec((n, f), lambda k, r: (0, 0)),
                  pl.BlockSpec((n, 1), lambda k, r: (0, 0))],
        out_specs=pl.BlockSpec((n, f), lambda k, r: (0, 0)),
        out_shape=jax.ShapeDtypeStruct((n, f), jnp.float32),
        scratch_shapes=[pltpu.VMEM((n, f), jnp.float32),
                        pltpu.VMEM((n, f), jnp.float32),
                        pltpu.VMEM((n, f), jnp.float32),
                        pltpu.VMEM((n, f), jnp.bfloat16)],
        compiler_params=pltpu.CompilerParams(
            dimension_semantics=("arbitrary", "arbitrary")),
    )(coe, abf, x, dinv)
    return out


# speedup vs baseline: 3.4234x; 1.0272x over previous
import functools

import jax
import jax.numpy as jnp
from jax import lax
from jax.experimental import pallas as pl
from jax.experimental.pallas import tpu as pltpu


def _pick_block(n, candidates):
    for c in candidates:
        if n % c == 0:
            return c
    return n


def _cast_deg_body(a_ref, abf_ref, dinv_ref):
    a = a_ref[...]
    abf_ref[...] = a.astype(jnp.float8_e4m3fn)
    deg = jnp.sum(a, axis=1, keepdims=True)
    dinv_ref[...] = jnp.where(deg > 0.0, lax.rsqrt(deg), 0.0)


def _mlp_body(f_ref, w1_ref, b1_ref, w2_ref, b2_ref, x_ref):
    h = jnp.dot(f_ref[...], w1_ref[...], preferred_element_type=jnp.float32)
    h = jnp.maximum(h + b1_ref[...], 0.0)
    x_ref[...] = (
        jnp.dot(h, w2_ref[...], preferred_element_type=jnp.float32) + b2_ref[...]
    )


def _cheb_body(nb, coe_ref, abf_ref, x_ref, dinv_ref, out_ref,
               acc_ref, tx0_ref, tx1_ref, ybf_ref):
    k = pl.program_id(0)
    rb = pl.program_id(1)

    @pl.when(rb == 0)
    def _prologue():
        @pl.when(k == 0)
        def _init():
            tx0_ref[...] = x_ref[...]
            out_ref[...] = (coe_ref[0] * 0.5) * x_ref[...]

        h = jnp.where(k == 0, x_ref[...], tx1_ref[...])
        ybf_ref[...] = (dinv_ref[...] * h).astype(jnp.bfloat16)

    rbsz = abf_ref.shape[0]
    yblk = ybf_ref[pl.ds(rb * rbsz, rbsz), :]
    contrib = lax.dot_general(
        abf_ref[...], yblk, (((0,), (0,)), ((), ())),
        preferred_element_type=jnp.float32)

    @pl.when(rb == 0)
    def _acc_init():
        acc_ref[...] = contrib

    @pl.when(rb > 0)
    def _acc_add():
        acc_ref[...] = acc_ref[...] + contrib

    @pl.when(rb == nb - 1)
    def _epilogue():
        prop = -dinv_ref[...] * acc_ref[...]
        txn = jnp.where(k == 0, prop, 2.0 * prop - tx0_ref[...])
        out_ref[...] = out_ref[...] + coe_ref[k + 1] * txn
        tx0_ref[...] = jnp.where(k == 0, x_ref[...], tx1_ref[...])
        tx1_ref[...] = txn


@jax.jit
def kernel(feature, A, W1, b1, W2, b2, temp):
    n = A.shape[0]
    nfeat = feature.shape[1]
    nhid = W1.shape[1]
    f = W2.shape[1]
    kk = temp.shape[0] - 1

    ct = jax.nn.relu(temp)
    j = jnp.arange(kk + 1, dtype=jnp.float32)
    xj = jnp.cos((kk - j + 0.5) * jnp.pi / (kk + 1))
    i = jnp.arange(kk + 1, dtype=jnp.float32)
    tcheb = jnp.cos(i[:, None] * jnp.arccos(xj)[None, :])
    coe = (2.0 / (kk + 1)) * (tcheb @ ct)

    rb1 = _pick_block(n, (200, 100, 50, 40, 25, 10, 8))
    abf, dinv = pl.pallas_call(
        _cast_deg_body,
        grid=(n // rb1,),
        in_specs=[pl.BlockSpec((rb1, n), lambda i: (i, 0))],
        out_specs=[pl.BlockSpec((rb1, n), lambda i: (i, 0)),
                   pl.BlockSpec((rb1, 1), lambda i: (i, 0))],
        out_shape=[jax.ShapeDtypeStruct((n, n), jnp.float8_e4m3fn),
                   jax.ShapeDtypeStruct((n, 1), jnp.float32)],
    )(A)

    mb = _pick_block(n, (1000, 500, 200, 100, 50, 10, 8))
    x = pl.pallas_call(
        _mlp_body,
        grid=(n // mb,),
        in_specs=[pl.BlockSpec((mb, nfeat), lambda i: (i, 0)),
                  pl.BlockSpec((nfeat, nhid), lambda i: (0, 0)),
                  pl.BlockSpec((1, nhid), lambda i: (0, 0)),
                  pl.BlockSpec((nhid, f), lambda i: (0, 0)),
                  pl.BlockSpec((1, f), lambda i: (0, 0))],
        out_specs=pl.BlockSpec((mb, f), lambda i: (i, 0)),
        out_shape=jax.ShapeDtypeStruct((n, f), jnp.float32),
    )(feature, W1, b1.reshape(1, nhid), W2, b2.reshape(1, f))

    rb = _pick_block(n, (400, 200, 100, 50, 40, 25, 10, 8))
    nb = n // rb
    out = pl.pallas_call(
        functools.partial(_cheb_body, nb),
        grid=(kk, nb),
        in_specs=[pl.BlockSpec(memory_space=pltpu.SMEM),
                  pl.BlockSpec((rb, n), lambda k, r: (r, 0)),
                  pl.BlockSpec((n, f), lambda k, r: (0, 0)),
                  pl.BlockSpec((n, 1), lambda k, r: (0, 0))],
        out_specs=pl.BlockSpec((n, f), lambda k, r: (0, 0)),
        out_shape=jax.ShapeDtypeStruct((n, f), jnp.float32),
        scratch_shapes=[pltpu.VMEM((n, f), jnp.float32),
                        pltpu.VMEM((n, f), jnp.float32),
                        pltpu.VMEM((n, f), jnp.float32),
                        pltpu.VMEM((n, f), jnp.bfloat16)],
        compiler_params=pltpu.CompilerParams(
            dimension_semantics=("arbitrary", "arbitrary")),
    )(coe, abf, x, dinv)
    return out


# speedup vs baseline: 5.6334x; 1.6455x over previous
import functools

import jax
import jax.numpy as jnp
from jax import lax
from jax.experimental import pallas as pl
from jax.experimental.pallas import tpu as pltpu


def _pick_block(n, candidates):
    for c in candidates:
        if n % c == 0:
            return c
    return n


def _cast_deg_body(a_ref, abf_ref, dinv_ref):
    a = a_ref[...]
    abf_ref[...] = a.astype(jnp.float8_e4m3fn)
    deg = jnp.sum(a, axis=1, keepdims=True)
    dinv_ref[...] = jnp.where(deg > 0.0, lax.rsqrt(deg), 0.0)


def _mlp_body(f_ref, w1_ref, b1_ref, w2_ref, b2_ref, x_ref):
    h = jnp.dot(f_ref[...], w1_ref[...], preferred_element_type=jnp.float32)
    h = jnp.maximum(h + b1_ref[...], 0.0)
    x_ref[...] = (
        jnp.dot(h, w2_ref[...], preferred_element_type=jnp.float32) + b2_ref[...]
    )


def _cheb_body(n, coe_ref, abf_ref, x_ref, dinv_ref, out_ref,
               tx0_ref, tx1_ref, ybt_ref):
    k = pl.program_id(0)
    cb = pl.program_id(1)

    @pl.when(cb == 0)
    def _prologue():
        @pl.when(k == 0)
        def _init():
            tx0_ref[...] = x_ref[...]
            out_ref[...] = (coe_ref[0] * 0.5) * x_ref[...]

        h = jnp.where(k == 0, x_ref[0:n, :], tx1_ref[0:n, :])
        y = (dinv_ref[0:n, :] * h).astype(jnp.bfloat16)
        ybt_ref[...] = jnp.swapaxes(y, 0, 1)

    cbsz = abf_ref.shape[1]
    t = lax.dot_general(
        ybt_ref[...], abf_ref[...], (((1,), (0,)), ((), ())),
        preferred_element_type=jnp.float32)
    tt = jnp.swapaxes(t, 0, 1)

    ds = pl.ds(cb * cbsz, cbsz)
    prop = -dinv_ref[ds, :] * tt
    txn = jnp.where(k == 0, prop, 2.0 * prop - tx0_ref[ds, :])
    out_ref[ds, :] = out_ref[ds, :] + coe_ref[k + 1] * txn
    tx0_ref[ds, :] = jnp.where(k == 0, x_ref[ds, :], tx1_ref[ds, :])
    tx1_ref[ds, :] = txn


@jax.jit
def kernel(feature, A, W1, b1, W2, b2, temp):
    n = A.shape[0]
    nfeat = feature.shape[1]
    nhid = W1.shape[1]
    f = W2.shape[1]
    kk = temp.shape[0] - 1

    ct = jax.nn.relu(temp)
    j = jnp.arange(kk + 1, dtype=jnp.float32)
    xj = jnp.cos((kk - j + 0.5) * jnp.pi / (kk + 1))
    i = jnp.arange(kk + 1, dtype=jnp.float32)
    tcheb = jnp.cos(i[:, None] * jnp.arccos(xj)[None, :])
    coe = (2.0 / (kk + 1)) * (tcheb @ ct)

    cbsz = 512
    npad = ((n + cbsz - 1) // cbsz) * cbsz
    nb = npad // cbsz

    rb1 = _pick_block(n, (200, 100, 50, 40, 25, 10, 8))
    abf, dinv = pl.pallas_call(
        _cast_deg_body,
        grid=(n // rb1,),
        in_specs=[pl.BlockSpec((rb1, n), lambda i: (i, 0))],
        out_specs=[pl.BlockSpec((rb1, n), lambda i: (i, 0)),
                   pl.BlockSpec((rb1, 1), lambda i: (i, 0))],
        out_shape=[jax.ShapeDtypeStruct((n, n), jnp.float8_e4m3fn),
                   jax.ShapeDtypeStruct((npad, 1), jnp.float32)],
    )(A)

    mb = _pick_block(npad, (1024, 512, 256, 128, 64, 32, 16, 8))
    x = pl.pallas_call(
        _mlp_body,
        grid=(npad // mb,),
        in_specs=[pl.BlockSpec((mb, nfeat), lambda i: (i, 0)),
                  pl.BlockSpec((nfeat, nhid), lambda i: (0, 0)),
                  pl.BlockSpec((1, nhid), lambda i: (0, 0)),
                  pl.BlockSpec((nhid, f), lambda i: (0, 0)),
                  pl.BlockSpec((1, f), lambda i: (0, 0))],
        out_specs=pl.BlockSpec((mb, f), lambda i: (i, 0)),
        out_shape=jax.ShapeDtypeStruct((npad, f), jnp.float32),
    )(feature, W1, b1.reshape(1, nhid), W2, b2.reshape(1, f))

    out = pl.pallas_call(
        functools.partial(_cheb_body, n),
        grid=(kk, nb),
        in_specs=[pl.BlockSpec(memory_space=pltpu.SMEM),
                  pl.BlockSpec((n, cbsz), lambda k, c: (0, c)),
                  pl.BlockSpec((npad, f), lambda k, c: (0, 0)),
                  pl.BlockSpec((npad, 1), lambda k, c: (0, 0))],
        out_specs=pl.BlockSpec((npad, f), lambda k, c: (0, 0)),
        out_shape=jax.ShapeDtypeStruct((npad, f), jnp.float32),
        scratch_shapes=[pltpu.VMEM((npad, f), jnp.float32),
                        pltpu.VMEM((npad, f), jnp.float32),
                        pltpu.VMEM((f, n), jnp.bfloat16)],
        compiler_params=pltpu.CompilerParams(
            dimension_semantics=("arbitrary", "arbitrary")),
    )(coe, abf, x, dinv)
    return out[:n]


# speedup vs baseline: 6.2232x; 1.1047x over previous
import functools

import jax
import jax.numpy as jnp
from jax import lax
from jax.experimental import pallas as pl
from jax.experimental.pallas import tpu as pltpu


def _pick_block(n, candidates):
    for c in candidates:
        if n % c == 0:
            return c
    return n


def _cast_deg_body(a_ref, abf_ref, dinv_ref):
    a = a_ref[...]
    abf_ref[...] = a.astype(jnp.float8_e4m3fn)
    deg = jnp.sum(a, axis=1, keepdims=True)
    dinv_ref[...] = jnp.where(deg > 0.0, lax.rsqrt(deg), 0.0)


def _mlp_body(f_ref, w1_ref, b1_ref, w2_ref, b2_ref, x_ref):
    h = jnp.dot(f_ref[...], w1_ref[...], preferred_element_type=jnp.float32)
    h = jnp.maximum(h + b1_ref[...], 0.0)
    x_ref[...] = (
        jnp.dot(h, w2_ref[...], preferred_element_type=jnp.float32) + b2_ref[...]
    )


def _cheb_body(n, coe_ref, abf_ref, x_ref, dinv_ref, out_ref,
               tx0_ref, tx1_ref, ybt_ref):
    k = pl.program_id(0)
    cb = pl.program_id(1)

    @pl.when(cb == 0)
    def _prologue():
        @pl.when(k == 0)
        def _init():
            tx0_ref[...] = x_ref[...]
            out_ref[...] = (coe_ref[0] * 0.5) * x_ref[...]

        h = jnp.where(k == 0, x_ref[0:n, :], tx1_ref[0:n, :])
        y = (dinv_ref[0:n, :] * h).astype(jnp.bfloat16)
        ybt_ref[...] = jnp.swapaxes(y, 0, 1)

    cbsz = abf_ref.shape[1]
    t = lax.dot_general(
        ybt_ref[...], abf_ref[...], (((1,), (0,)), ((), ())),
        preferred_element_type=jnp.float32)
    tt = jnp.swapaxes(t, 0, 1)

    ds = pl.ds(cb * cbsz, cbsz)
    prop = -dinv_ref[ds, :] * tt
    txn = jnp.where(k == 0, prop, 2.0 * prop - tx0_ref[ds, :])
    out_ref[ds, :] = out_ref[ds, :] + coe_ref[k + 1] * txn
    tx0_ref[ds, :] = jnp.where(k == 0, x_ref[ds, :], tx1_ref[ds, :])
    tx1_ref[ds, :] = txn


@jax.jit
def kernel(feature, A, W1, b1, W2, b2, temp):
    n = A.shape[0]
    nfeat = feature.shape[1]
    nhid = W1.shape[1]
    f = W2.shape[1]
    kk = temp.shape[0] - 1

    ct = jax.nn.relu(temp)
    j = jnp.arange(kk + 1, dtype=jnp.float32)
    xj = jnp.cos((kk - j + 0.5) * jnp.pi / (kk + 1))
    i = jnp.arange(kk + 1, dtype=jnp.float32)
    tcheb = jnp.cos(i[:, None] * jnp.arccos(xj)[None, :])
    coe = (2.0 / (kk + 1)) * (tcheb @ ct)

    cbsz = 1024
    npad = ((n + cbsz - 1) // cbsz) * cbsz
    nb = npad // cbsz

    rb1 = _pick_block(n, (200, 100, 50, 40, 25, 10, 8))
    abf, dinv = pl.pallas_call(
        _cast_deg_body,
        grid=(n // rb1,),
        in_specs=[pl.BlockSpec((rb1, n), lambda i: (i, 0))],
        out_specs=[pl.BlockSpec((rb1, n), lambda i: (i, 0)),
                   pl.BlockSpec((rb1, 1), lambda i: (i, 0))],
        out_shape=[jax.ShapeDtypeStruct((n, n), jnp.float8_e4m3fn),
                   jax.ShapeDtypeStruct((npad, 1), jnp.float32)],
    )(A)

    mb = _pick_block(npad, (1024, 512, 256, 128, 64, 32, 16, 8))
    x = pl.pallas_call(
        _mlp_body,
        grid=(npad // mb,),
        in_specs=[pl.BlockSpec((mb, nfeat), lambda i: (i, 0)),
                  pl.BlockSpec((nfeat, nhid), lambda i: (0, 0)),
                  pl.BlockSpec((1, nhid), lambda i: (0, 0)),
                  pl.BlockSpec((nhid, f), lambda i: (0, 0)),
                  pl.BlockSpec((1, f), lambda i: (0, 0))],
        out_specs=pl.BlockSpec((mb, f), lambda i: (i, 0)),
        out_shape=jax.ShapeDtypeStruct((npad, f), jnp.float32),
    )(feature, W1, b1.reshape(1, nhid), W2, b2.reshape(1, f))

    out = pl.pallas_call(
        functools.partial(_cheb_body, n),
        grid=(kk, nb),
        in_specs=[pl.BlockSpec(memory_space=pltpu.SMEM),
                  pl.BlockSpec((n, cbsz), lambda k, c: (0, c)),
                  pl.BlockSpec((npad, f), lambda k, c: (0, 0)),
                  pl.BlockSpec((npad, 1), lambda k, c: (0, 0))],
        out_specs=pl.BlockSpec((npad, f), lambda k, c: (0, 0)),
        out_shape=jax.ShapeDtypeStruct((npad, f), jnp.float32),
        scratch_shapes=[pltpu.VMEM((npad, f), jnp.float32),
                        pltpu.VMEM((npad, f), jnp.float32),
                        pltpu.VMEM((f, n), jnp.bfloat16)],
        compiler_params=pltpu.CompilerParams(
            dimension_semantics=("arbitrary", "arbitrary")),
    )(coe, abf, x, dinv)
    return out[:n]


# speedup vs baseline: 6.4202x; 1.0317x over previous
import functools

import jax
import jax.numpy as jnp
from jax import lax
from jax.experimental import pallas as pl
from jax.experimental.pallas import tpu as pltpu


def _pick_block(n, candidates):
    for c in candidates:
        if n % c == 0:
            return c
    return n


def _front_body(n, npad, nblk,
                a_ref, f_ref, w1_ref, b1_ref, w2_ref, b2_ref,
                a8_ref, dinv_ref, x_ref, t1_ref,
                acct_ref, dinvs_ref):
    rb = pl.program_id(0)

    a = a_ref[...]
    a8_ref[...] = a.astype(jnp.float8_e4m3fn)

    h = jnp.dot(f_ref[...], w1_ref[...], preferred_element_type=jnp.float32)
    h = jnp.maximum(h + b1_ref[...], 0.0)
    x = jnp.dot(h, w2_ref[...], preferred_element_type=jnp.float32) + b2_ref[...]
    x_ref[...] = x

    deg = jnp.sum(a, axis=1, keepdims=True)
    dinv = jnp.where(deg > 0.0, lax.rsqrt(deg), 0.0)
    dinv_ref[...] = dinv

    rbsz = a.shape[0]
    dinvs_ref[pl.ds(rb * rbsz, rbsz), :] = dinv

    y = (dinv * x).astype(jnp.bfloat16)
    ab = a.astype(jnp.bfloat16)
    contrib = lax.dot_general(
        y, ab, (((0,), (0,)), ((), ())), preferred_element_type=jnp.float32)

    @pl.when(rb == 0)
    def _acc_init():
        acct_ref[...] = contrib

    @pl.when(rb > 0)
    def _acc_add():
        acct_ref[...] = acct_ref[...] + contrib

    @pl.when(rb == nblk - 1)
    def _finalize():
        t1 = jnp.swapaxes(acct_ref[...], 0, 1)
        t1_ref[0:n, :] = -dinvs_ref[0:n, :] * t1
        t1_ref[n:npad, :] = jnp.zeros((npad - n, t1.shape[1]), jnp.float32)


def _cheb_body(n, coe_ref, abf_ref, x_ref, dinv_ref, t1_ref, out_ref,
               tx0_ref, tx1_ref, ybt_ref):
    k = pl.program_id(0)
    cb = pl.program_id(1)

    @pl.when(cb == 0)
    def _prologue():
        @pl.when(k == 0)
        def _init():
            tx0_ref[...] = x_ref[...]
            tx1_ref[...] = t1_ref[...]
            out_ref[...] = (coe_ref[0] * 0.5) * x_ref[...] \
                + coe_ref[1] * t1_ref[...]

        y = (dinv_ref[0:n, :] * tx1_ref[0:n, :]).astype(jnp.bfloat16)
        ybt_ref[...] = jnp.swapaxes(y, 0, 1)

    cbsz = abf_ref.shape[1]
    t = lax.dot_general(
        ybt_ref[...], abf_ref[...], (((1,), (0,)), ((), ())),
        preferred_element_type=jnp.float32)
    tt = jnp.swapaxes(t, 0, 1)

    ds = pl.ds(cb * cbsz, cbsz)
    prop = -dinv_ref[ds, :] * tt
    txn = 2.0 * prop - tx0_ref[ds, :]
    out_ref[ds, :] = out_ref[ds, :] + coe_ref[k + 2] * txn
    tx0_ref[ds, :] = tx1_ref[ds, :]
    tx1_ref[ds, :] = txn


@jax.jit
def kernel(feature, A, W1, b1, W2, b2, temp):
    n = A.shape[0]
    nfeat = feature.shape[1]
    nhid = W1.shape[1]
    f = W2.shape[1]
    kk = temp.shape[0] - 1

    ct = jax.nn.relu(temp)
    j = jnp.arange(kk + 1, dtype=jnp.float32)
    xj = jnp.cos((kk - j + 0.5) * jnp.pi / (kk + 1))
    i = jnp.arange(kk + 1, dtype=jnp.float32)
    tcheb = jnp.cos(i[:, None] * jnp.arccos(xj)[None, :])
    coe = (2.0 / (kk + 1)) * (tcheb @ ct)

    cbsz = 1024
    npad = ((n + cbsz - 1) // cbsz) * cbsz
    nb = npad // cbsz

    rb1 = _pick_block(n, (200, 100, 50, 40, 25, 10, 8))
    nblk = n // rb1
    a8, dinv, x, t1 = pl.pallas_call(
        functools.partial(_front_body, n, npad, nblk),
        grid=(nblk,),
        in_specs=[pl.BlockSpec((rb1, n), lambda i: (i, 0)),
                  pl.BlockSpec((rb1, nfeat), lambda i: (i, 0)),
                  pl.BlockSpec((nfeat, nhid), lambda i: (0, 0)),
                  pl.BlockSpec((1, nhid), lambda i: (0, 0)),
                  pl.BlockSpec((nhid, f), lambda i: (0, 0)),
                  pl.BlockSpec((1, f), lambda i: (0, 0))],
        out_specs=[pl.BlockSpec((rb1, n), lambda i: (i, 0)),
                   pl.BlockSpec((rb1, 1), lambda i: (i, 0)),
                   pl.BlockSpec((rb1, f), lambda i: (i, 0)),
                   pl.BlockSpec((npad, f), lambda i: (0, 0))],
        out_shape=[jax.ShapeDtypeStruct((n, n), jnp.float8_e4m3fn),
                   jax.ShapeDtypeStruct((npad, 1), jnp.float32),
                   jax.ShapeDtypeStruct((npad, f), jnp.float32),
                   jax.ShapeDtypeStruct((npad, f), jnp.float32)],
        scratch_shapes=[pltpu.VMEM((f, n), jnp.float32),
                        pltpu.VMEM((npad, 1), jnp.float32)],
        compiler_params=pltpu.CompilerParams(
            dimension_semantics=("arbitrary",)),
    )(A, feature, W1, b1.reshape(1, nhid), W2, b2.reshape(1, f))

    out = pl.pallas_call(
        functools.partial(_cheb_body, n),
        grid=(kk - 1, nb),
        in_specs=[pl.BlockSpec(memory_space=pltpu.SMEM),
                  pl.BlockSpec((n, cbsz), lambda k, c: (0, c)),
                  pl.BlockSpec((npad, f), lambda k, c: (0, 0)),
                  pl.BlockSpec((npad, 1), lambda k, c: (0, 0)),
                  pl.BlockSpec((npad, f), lambda k, c: (0, 0))],
        out_specs=pl.BlockSpec((npad, f), lambda k, c: (0, 0)),
        out_shape=jax.ShapeDtypeStruct((npad, f), jnp.float32),
        scratch_shapes=[pltpu.VMEM((npad, f), jnp.float32),
                        pltpu.VMEM((npad, f), jnp.float32),
                        pltpu.VMEM((f, n), jnp.bfloat16)],
        compiler_params=pltpu.CompilerParams(
            dimension_semantics=("arbitrary", "arbitrary")),
    )(coe, a8, x, dinv, t1)
    return out[:n]
